# baseline (speedup 1.0000x reference)
import functools

import jax
import jax.numpy as jnp
from jax import lax
from jax.experimental import pallas as pl
from jax.experimental.pallas import tpu as pltpu
from jax.experimental.pallas import tpu_sc as plsc

NUM_EMB = 128
MAX_POS = 16
NBINS = NUM_EMB + MAX_POS
LANES = 16
CHARS = 16
BATCH = 16
WORDS = 1024
NC, NS = 2, 16
NW = NC * NS
BT = NBINS // 8
WT = WORDS // 128
WTH = WT // 2

_MESH = plsc.VectorSubcoreMesh(
    core_axis_name="c", subcore_axis_name="s", num_cores=NC, num_subcores=NS
)


@functools.partial(
    pl.kernel,
    out_type=jax.ShapeDtypeStruct((BATCH, BT, WT, 8, 128), jnp.int32),
    mesh=_MESH,
    scratch_types=[
        pltpu.VMEM((2, WTH, 8, 128), jnp.int32),
        pltpu.VMEM((2, WTH, 8, 128), jnp.int32),
        pltpu.VMEM((BT, WTH, 8, 128), jnp.int32),
        pltpu.SemaphoreType.DMA,
        pltpu.SemaphoreType.DMA,
    ],
    compiler_params=pltpu.CompilerParams(needs_layout_passes=False),
)
def _sc_encode(tok_hbm, pos_hbm, out_hbm, tok_v, pos_v, out_v, sem_in, sem_out):
    wid = lax.axis_index("s") * NC + lax.axis_index("c")
    b = wid // 2
    wt0 = (wid % 2) * WTH

    cp_t = pltpu.async_copy(tok_hbm.at[b, :, pl.ds(wt0, WTH)], tok_v, sem_in)
    cp_p = pltpu.async_copy(pos_hbm.at[b, :, pl.ds(wt0, WTH)], pos_v, sem_in)
    cp_t.wait()
    cp_p.wait()

    zeros = jnp.zeros((LANES,), jnp.int32)
    ones = jnp.full((LANES,), 1, jnp.int32)
    iota = jax.lax.iota(jnp.int32, LANES)

    out_cps = []
    for wt in range(WTH):
        wt_vec = jnp.full((LANES,), wt, jnp.int32)

        @plsc.parallel_loop(0, 8, step=1, unroll=1)
        def group(g):
            l0 = g * LANES
            lanes = l0 + iota
            for bt in range(BT):
                for s in range(8):
                    out_v[bt, wt, s, pl.ds(l0, LANES)] = zeros
            for c in range(CHARS):
                tok = tok_v[c // 8, wt, c % 8, pl.ds(l0, LANES)]
                plsc.addupdate_scatter(
                    out_v,
                    [tok >> 3, wt_vec, tok & 7, lanes],
                    ones,
                    mask=tok != 0,
                )
                pos = pos_v[c // 8, wt, c % 8, pl.ds(l0, LANES)]
                plsc.addupdate_scatter(
                    out_v,
                    [(NUM_EMB + pos) >> 3, wt_vec, pos & 7, lanes],
                    ones,
                    mask=pos != 0,
                )

        out_cps.append(
            pltpu.async_copy(
                out_v.at[:, wt], out_hbm.at[b, :, wt0 + wt], sem_out
            )
        )

    for cp in out_cps:
        cp.wait()


def kernel(token_ids, position_ids):
    def to_tiles(x):
        x = x.transpose(0, 2, 1).reshape(BATCH, 2, 8, WT, 128)
        return x.transpose(0, 1, 3, 2, 4)

    out = _sc_encode(to_tiles(token_ids), to_tiles(position_ids))
    out = out.transpose(0, 1, 3, 2, 4).reshape(BATCH, NBINS, WORDS)
    return out.transpose(0, 2, 1)


# speedup vs baseline: 1.0513x; 1.0513x over previous
import functools

import jax
import jax.numpy as jnp
from jax import lax
from jax.experimental import pallas as pl
from jax.experimental.pallas import tpu as pltpu
from jax.experimental.pallas import tpu_sc as plsc

NUM_EMB = 128
MAX_POS = 16
NBINS = NUM_EMB + MAX_POS
LANES = 16
CHARS = 16
BATCH = 16
WORDS = 1024
NC, NS = 2, 16
NW = NC * NS
BT = NBINS // 8
WT = WORDS // 128
WTH = WT // 2

_MESH = plsc.VectorSubcoreMesh(
    core_axis_name="c", subcore_axis_name="s", num_cores=NC, num_subcores=NS
)


@functools.partial(
    pl.kernel,
    out_type=jax.ShapeDtypeStruct((BATCH, BT, WT, 8, 128), jnp.int32),
    mesh=_MESH,
    scratch_types=[
        pltpu.VMEM((2, WTH, 8, 128), jnp.int32),
        pltpu.VMEM((2, WTH, 8, 128), jnp.int32),
        pltpu.VMEM((BT, 8, 128), jnp.int32),
        pltpu.VMEM((BT, 8, 128), jnp.int32),
        pltpu.SemaphoreType.DMA,
        pltpu.SemaphoreType.DMA,
        pltpu.SemaphoreType.DMA,
    ],
    compiler_params=pltpu.CompilerParams(needs_layout_passes=False),
)
def _sc_encode(
    tok_hbm, pos_hbm, out_hbm, tok_v, pos_v, out_a, out_b, sem_in, sem_a, sem_b
):
    wid = lax.axis_index("s") * NC + lax.axis_index("c")
    b = wid // 2
    wt0 = (wid % 2) * WTH

    cp_t = pltpu.async_copy(tok_hbm.at[b, :, pl.ds(wt0, WTH)], tok_v, sem_in)
    cp_p = pltpu.async_copy(pos_hbm.at[b, :, pl.ds(wt0, WTH)], pos_v, sem_in)
    cp_t.wait()
    cp_p.wait()

    zeros = jnp.zeros((LANES,), jnp.int32)
    ones = jnp.full((LANES,), 1, jnp.int32)
    iota = jax.lax.iota(jnp.int32, LANES)

    out_cps = [None, None]
    for wt in range(WTH):
        out_v = out_a if wt % 2 == 0 else out_b
        if out_cps[wt % 2] is not None:
            out_cps[wt % 2].wait()

        @plsc.parallel_loop(0, 8, step=1, unroll=1)
        def group(g):
            l0 = g * LANES
            lanes = l0 + iota
            for bt in range(BT):
                for s in range(8):
                    out_v[bt, s, pl.ds(l0, LANES)] = zeros
            for c in range(CHARS):
                tok = tok_v[c // 8, wt, c % 8, pl.ds(l0, LANES)]
                plsc.addupdate_scatter(
                    out_v,
                    [tok >> 3, tok & 7, lanes],
                    ones,
                    mask=tok != 0,
                )
                pos = pos_v[c // 8, wt, c % 8, pl.ds(l0, LANES)]
                plsc.addupdate_scatter(
                    out_v,
                    [(NUM_EMB + pos) >> 3, pos & 7, lanes],
                    ones,
                    mask=pos != 0,
                )

        out_cps[wt % 2] = pltpu.async_copy(
            out_v, out_hbm.at[b, :, wt0 + wt], sem_a if wt % 2 == 0 else sem_b
        )

    for cp in out_cps:
        cp.wait()


def kernel(token_ids, position_ids):
    def to_tiles(x):
        x = x.transpose(0, 2, 1).reshape(BATCH, 2, 8, WT, 128)
        return x.transpose(0, 1, 3, 2, 4)

    out = _sc_encode(to_tiles(token_ids), to_tiles(position_ids))
    out = out.transpose(0, 1, 3, 2, 4).reshape(BATCH, NBINS, WORDS)
    return out.transpose(0, 2, 1)
